# Initial kernel scaffold; baseline (speedup 1.0000x reference)
import jax
import jax.numpy as jnp
from jax.experimental import pallas as pl


def kernel(words_as_ids, embedding_weight):
    raise NotImplementedError("write your pallas kernel here")


# speedup vs baseline: 1.4764x; 1.4764x over previous
import functools

import jax
import jax.numpy as jnp
from jax import lax
from jax.experimental import pallas as pl
from jax.experimental.pallas import tpu as pltpu
from jax.experimental.pallas import tpu_sc as plsc

NUM_CORES = 2
NUM_SUBCORES = 16
NUM_WORKERS = NUM_CORES * NUM_SUBCORES
CHUNK = 128
GROUP = 8


@functools.partial(jax.jit, static_argnums=())
def _sc_embedding_gather(idx2d, table):
    rows, chunk = idx2d.shape
    vocab, dim = table.shape
    rows_per_w = rows // NUM_WORKERS
    n_groups = rows_per_w // GROUP
    mesh = plsc.VectorSubcoreMesh(core_axis_name="c", subcore_axis_name="s")

    @functools.partial(
        pl.kernel,
        mesh=mesh,
        out_type=jax.ShapeDtypeStruct((rows * chunk, dim), jnp.float32),
        compiler_params=pltpu.CompilerParams(use_tc_tiling_on_sc=False),
        scratch_types=[
            pltpu.VMEM((rows_per_w, chunk), jnp.int32),
            pltpu.VMEM((GROUP * chunk, dim), jnp.float32),
            pltpu.SemaphoreType.DMA,
        ],
    )
    def k(idx_hbm, table_hbm, out_hbm, idx_v, rows_v, sem):
        wid = lax.axis_index("s") * NUM_CORES + lax.axis_index("c")
        row0 = wid * rows_per_w
        pltpu.sync_copy(idx_hbm.at[pl.ds(row0, rows_per_w)], idx_v)

        def body(g, carry):
            copies = []
            for j in range(GROUP):
                copies.append(
                    pltpu.async_copy(
                        table_hbm.at[idx_v.at[g * GROUP + j]],
                        rows_v.at[pl.ds(j * CHUNK, CHUNK)],
                        sem,
                    )
                )
            for c in copies:
                c.wait()
            out_base = (row0 + g * GROUP) * CHUNK
            pltpu.sync_copy(rows_v, out_hbm.at[pl.ds(out_base, GROUP * CHUNK)])
            return carry

        lax.fori_loop(0, n_groups, body, 0)

    return k(idx2d, table)


def kernel(words_as_ids, embedding_weight):
    batch, hist = words_as_ids.shape
    dim = embedding_weight.shape[1]
    idx2d = words_as_ids.reshape(-1, CHUNK)
    out = _sc_embedding_gather(idx2d, embedding_weight)
    return out.reshape(batch, hist, dim)


# speedup vs baseline: 1.5000x; 1.0160x over previous
import functools

import jax
import jax.numpy as jnp
from jax import lax
from jax.experimental import pallas as pl
from jax.experimental.pallas import tpu as pltpu
from jax.experimental.pallas import tpu_sc as plsc

NUM_CORES = 2
NUM_SUBCORES = 16
NUM_WORKERS = NUM_CORES * NUM_SUBCORES
CHUNK = 128
GROUP = 10


@jax.jit
def _sc_embedding_gather(idx2d, table):
    rows, chunk = idx2d.shape
    vocab, dim = table.shape
    rows_per_w = rows // NUM_WORKERS
    n_groups = rows_per_w // GROUP
    assert n_groups % 2 == 0 and n_groups * GROUP == rows_per_w
    mesh = plsc.VectorSubcoreMesh(core_axis_name="c", subcore_axis_name="s")

    @functools.partial(
        pl.kernel,
        mesh=mesh,
        out_type=jax.ShapeDtypeStruct((rows * chunk, dim), jnp.float32),
        compiler_params=pltpu.CompilerParams(use_tc_tiling_on_sc=False),
        scratch_types=[
            pltpu.VMEM((rows_per_w, chunk), jnp.int32),
            pltpu.VMEM((GROUP * chunk, dim), jnp.float32),
            pltpu.VMEM((GROUP * chunk, dim), jnp.float32),
            pltpu.SemaphoreType.DMA,
            pltpu.SemaphoreType.DMA,
        ],
    )
    def k(idx_hbm, table_hbm, out_hbm, idx_v, buf0, buf1, sem0, sem1):
        wid = lax.axis_index("s") * NUM_CORES + lax.axis_index("c")
        row0 = wid * rows_per_w
        pltpu.sync_copy(idx_hbm.at[pl.ds(row0, rows_per_w)], idx_v)

        def fire(g, buf, sem):
            for j in range(GROUP):
                pltpu.async_copy(
                    table_hbm.at[idx_v.at[g * GROUP + j]],
                    buf.at[pl.ds(j * CHUNK, CHUNK)],
                    sem,
                )

        def drain(buf, sem):
            for j in range(GROUP):
                pltpu.make_async_copy(
                    table_hbm.at[pl.ds(0, CHUNK)],
                    buf.at[pl.ds(j * CHUNK, CHUNK)],
                    sem,
                ).wait()

        def store(g, buf):
            out_base = (row0 + g * GROUP) * CHUNK
            pltpu.sync_copy(buf, out_hbm.at[pl.ds(out_base, GROUP * CHUNK)])

        fire(0, buf0, sem0)

        def body(t2, carry):
            g = 2 * t2
            fire(g + 1, buf1, sem1)
            drain(buf0, sem0)
            store(g, buf0)

            @pl.when(t2 < n_groups // 2 - 1)
            def _():
                fire(g + 2, buf0, sem0)

            drain(buf1, sem1)
            store(g + 1, buf1)
            return carry

        lax.fori_loop(0, n_groups // 2, body, 0)

    return k(idx2d, table)


def kernel(words_as_ids, embedding_weight):
    batch, hist = words_as_ids.shape
    dim = embedding_weight.shape[1]
    idx2d = words_as_ids.reshape(-1, CHUNK)
    out = _sc_embedding_gather(idx2d, embedding_weight)
    return out.reshape(batch, hist, dim)
